# Initial kernel scaffold; baseline (speedup 1.0000x reference)
import jax
import jax.numpy as jnp
from jax.experimental import pallas as pl


def kernel(x, u, i):
    raise NotImplementedError("write your pallas kernel here")


# speedup vs baseline: 8.0734x; 8.0734x over previous
import functools
import jax
import jax.numpy as jnp
from jax import lax
from jax.experimental import pallas as pl
from jax.experimental.pallas import tpu as pltpu

_N = 2000
_M = 8000
_D = 128
_ALPHA = 1.0
_CLAMP = 1e-06
_BJ = 1000
_JT = _M // _BJ

_F32 = jnp.float32
_BF = jnp.bfloat16


def _normalize(v):
    nrm = jnp.sqrt(jnp.sum(v * v, axis=1, keepdims=True))
    return v / jnp.maximum(nrm, 1e-12)


def _phase_a_body(c_ref, xit_ref, xu_ref,
                  outu_ref, wc_ref, ws_ref, g2_ref, aux_ref,
                  sxi, svi, accA, g1, dui8, csum, xun):
    j = pl.program_id(0)

    @pl.when(j == 0)
    def _init():
        sxi[...] = jnp.zeros_like(sxi)
        svi[...] = jnp.zeros_like(svi)
        accA[...] = jnp.zeros_like(accA)
        g1[...] = jnp.zeros_like(g1)
        dui8[...] = jnp.zeros_like(dui8)
        csum[...] = jnp.zeros_like(csum)
        xun[...] = _normalize(xu_ref[...])

    c = c_ref[...]
    vi_t = xit_ref[...]
    xi_t = _normalize(vi_t)
    xu_n = xun[...]

    cb = c.astype(_BF)
    xib = xi_t.astype(_BF)
    vib = vi_t.astype(_BF)
    p = jax.lax.dot_general(xib, xu_n.astype(_BF), (((1,), (1,)), ((), ())),
                            preferred_element_type=_F32)
    s = (c * p).astype(_BF)
    ct = (((0,), (0,)), ((), ()))
    sxi[...] += jax.lax.dot_general(cb, xib, ct, preferred_element_type=_F32)
    svi[...] += jax.lax.dot_general(cb, vib, ct, preferred_element_type=_F32)
    accA[...] += jax.lax.dot_general(s, vib, ct, preferred_element_type=_F32)
    g1[...] += jax.lax.dot_general(vib, xib, ct, preferred_element_type=_F32)
    ones8 = jnp.ones((_BJ, 8), _BF)
    dui8[...] += jax.lax.dot_general(cb, ones8, ct, preferred_element_type=_F32)
    csum[0:1, :] += jnp.sum(xi_t, axis=0, keepdims=True)
    csum[1:2, :] += jnp.sum(vi_t, axis=0, keepdims=True)

    @pl.when(j == _JT - 1)
    def _epilogue():
        xu_nv = xun[...]
        vu = xu_ref[...]
        dui_raw = dui8[...][:, 0:1]
        dui = jnp.maximum(dui_raw, 1.0)
        duj = jnp.maximum(jnp.asarray(_M, _F32) - dui_raw, 1.0)
        sxiv, sviv, av = sxi[...], svi[...], accA[...]
        sxj = csum[0:1, :] - sxiv
        svj = csum[1:2, :] - sviv
        b_pos = jnp.sum(xu_nv * sxiv, axis=1, keepdims=True) / dui
        b_neg = jnp.sum(xu_nv * sxj, axis=1, keepdims=True) / duj
        du = b_pos - b_neg + _ALPHA
        zu1 = av / dui - sviv * (b_neg - _ALPHA) / dui
        xg1 = jax.lax.dot_general(xu_nv.astype(_BF), g1[...].astype(_BF),
                                  (((1,), (0,)), ((), ())),
                                  preferred_element_type=_F32)
        zu2 = (xg1 - av) / duj - svj * (b_pos + _ALPHA) / duj
        dcl = jnp.maximum(du, _CLAMP)
        outu_ref[...] = (zu1 + zu2) / (dcl + dcl)

        w1 = xu_nv / dui
        w2 = xu_nv / duj
        w3 = vu * (b_neg - _ALPHA) / dui
        w4 = vu * (b_pos + _ALPHA) / duj
        w5 = vu / dui
        w6 = vu / duj
        s1 = (-b_neg + _ALPHA) / dui
        s2 = (b_pos + _ALPHA) / duj
        wc_ref[:, 0:128] = w1
        wc_ref[:, 128:256] = w2
        wc_ref[:, 256:384] = w3
        wc_ref[:, 384:512] = w4
        scal = jnp.concatenate(
            [s1, s2, jnp.zeros((_N, 126), _F32)], axis=1)
        wc_ref[:, 512:640] = scal
        ws_ref[:, 0:128] = w5
        ws_ref[:, 128:256] = w6
        g2_ref[...] = jax.lax.dot_general(w2.astype(_BF), vu.astype(_BF),
                                          (((0,), (0,)), ((), ())),
                                          preferred_element_type=_F32)
        aux_ref[...] = jnp.concatenate(
            [jnp.sum(w2, axis=0, keepdims=True),
             jnp.sum(w4, axis=0, keepdims=True),
             jnp.full((1, 128), jnp.sum(s2)),
             jnp.zeros((5, 128), _F32)], axis=0)


def _phase_b_body(c_ref, xit_ref, xu_ref, wc_ref, ws_ref, g2_ref, aux_ref,
                  outi_ref, xun):
    j = pl.program_id(0)

    @pl.when(j == 0)
    def _init():
        xun[...] = _normalize(xu_ref[...])

    c = c_ref[...]
    vi_t = xit_ref[...]
    xi_t = _normalize(vi_t)
    xu_n = xun[...]

    cb = c.astype(_BF)
    p = jax.lax.dot_general(xi_t.astype(_BF), xu_n.astype(_BF),
                            (((1,), (1,)), ((), ())),
                            preferred_element_type=_F32)
    s = (c * p).astype(_BF)
    nrm = (((1,), (0,)), ((), ()))
    segs = jax.lax.dot_general(cb, wc_ref[...].astype(_BF), nrm,
                               preferred_element_type=_F32)
    t56 = jax.lax.dot_general(s, ws_ref[...].astype(_BF), nrm,
                              preferred_element_type=_F32)
    segw1 = segs[:, 0:128]
    segw2 = segs[:, 128:256]
    segw3 = segs[:, 256:384]
    segw4 = segs[:, 384:512]
    segs1 = segs[:, 512:513]
    segs2 = segs[:, 513:514]
    t5 = t56[:, 0:128]
    t6 = t56[:, 128:256]
    aux = aux_ref[...]
    colsum_w2 = aux[0:1, :]
    colsum_w4 = aux[1:2, :]
    rest_b_tot = aux[2:3, 0:1]

    di1 = jnp.sum(xi_t * segw1, axis=1, keepdims=True) + segs1
    rest_x = colsum_w2 - segw2
    di2 = -jnp.sum(xi_t * rest_x, axis=1, keepdims=True) + (rest_b_tot - segs2)
    zi1 = t5 - segw3
    xg2 = jax.lax.dot_general(xi_t.astype(_BF), g2_ref[...].astype(_BF),
                              (((1,), (0,)), ((), ())),
                              preferred_element_type=_F32)
    zi2 = xg2 - t6 - (colsum_w4 - segw4)
    d1 = jnp.maximum(di1, _CLAMP)
    d2 = jnp.maximum(di2, _CLAMP)
    outi_ref[...] = (zi1 + zi2) / (d1 + d2)


def _densify_counts(u, i):
    return jnp.zeros((_M, _N), _F32).at[i, u].add(1.0)


def kernel(x, u, i):
    xu = x[:_N]
    xitems = x[_N:]
    c = _densify_counts(u, i)

    a_out = pl.pallas_call(
        _phase_a_body,
        grid=(_JT,),
        in_specs=[
            pl.BlockSpec((_BJ, _N), lambda j: (j, 0)),
            pl.BlockSpec((_BJ, _D), lambda j: (j, 0)),
            pl.BlockSpec((_N, _D), lambda j: (0, 0)),
        ],
        out_specs=[
            pl.BlockSpec((_N, _D), lambda j: (0, 0)),
            pl.BlockSpec((_N, 640), lambda j: (0, 0)),
            pl.BlockSpec((_N, 256), lambda j: (0, 0)),
            pl.BlockSpec((_D, _D), lambda j: (0, 0)),
            pl.BlockSpec((8, 128), lambda j: (0, 0)),
        ],
        out_shape=[
            jax.ShapeDtypeStruct((_N, _D), _F32),
            jax.ShapeDtypeStruct((_N, 640), _F32),
            jax.ShapeDtypeStruct((_N, 256), _F32),
            jax.ShapeDtypeStruct((_D, _D), _F32),
            jax.ShapeDtypeStruct((8, 128), _F32),
        ],
        scratch_shapes=[
            pltpu.VMEM((_N, _D), _F32),
            pltpu.VMEM((_N, _D), _F32),
            pltpu.VMEM((_N, _D), _F32),
            pltpu.VMEM((_D, _D), _F32),
            pltpu.VMEM((_N, 8), _F32),
            pltpu.VMEM((8, _D), _F32),
            pltpu.VMEM((_N, _D), _F32),
        ],
    )(c, xitems, xu)
    out_u, wc, ws, g2, aux = a_out

    out_i = pl.pallas_call(
        _phase_b_body,
        grid=(_JT,),
        in_specs=[
            pl.BlockSpec((_BJ, _N), lambda j: (j, 0)),
            pl.BlockSpec((_BJ, _D), lambda j: (j, 0)),
            pl.BlockSpec((_N, _D), lambda j: (0, 0)),
            pl.BlockSpec((_N, 640), lambda j: (0, 0)),
            pl.BlockSpec((_N, 256), lambda j: (0, 0)),
            pl.BlockSpec((_D, _D), lambda j: (0, 0)),
            pl.BlockSpec((8, 128), lambda j: (0, 0)),
        ],
        out_specs=pl.BlockSpec((_BJ, _D), lambda j: (j, 0)),
        out_shape=jax.ShapeDtypeStruct((_M, _D), _F32),
        scratch_shapes=[pltpu.VMEM((_N, _D), _F32)],
    )(c, xitems, xu, wc, ws, g2, aux)

    return jnp.concatenate([out_u, out_i], axis=0)


# speedup vs baseline: 10.1552x; 1.2579x over previous
import functools
import jax
import jax.numpy as jnp
from jax import lax
from jax.experimental import pallas as pl
from jax.experimental.pallas import tpu as pltpu
from jax.experimental.pallas import tpu_sc as plsc

_N = 2000
_M = 8000
_D = 128
_ALPHA = 1.0
_CLAMP = 1e-06
_BJ = 1000
_JT = _M // _BJ

_F32 = jnp.float32
_BF = jnp.bfloat16


def _normalize(v):
    nrm = jnp.sqrt(jnp.sum(v * v, axis=1, keepdims=True))
    return v / jnp.maximum(nrm, 1e-12)


def _phase_a_body(c_ref, xit_ref, xu_ref,
                  outu_ref, wc_ref, ws_ref, g2_ref, aux_ref,
                  sxi, svi, accA, g1, dui8, csum, xun):
    j = pl.program_id(0)

    @pl.when(j == 0)
    def _init():
        sxi[...] = jnp.zeros_like(sxi)
        svi[...] = jnp.zeros_like(svi)
        accA[...] = jnp.zeros_like(accA)
        g1[...] = jnp.zeros_like(g1)
        dui8[...] = jnp.zeros_like(dui8)
        csum[...] = jnp.zeros_like(csum)
        xun[...] = _normalize(xu_ref[...])

    c = c_ref[...]
    vi_t = xit_ref[...]
    xi_t = _normalize(vi_t)
    xu_n = xun[...]

    cb = c.astype(_BF)
    xib = xi_t.astype(_BF)
    vib = vi_t.astype(_BF)
    p = jax.lax.dot_general(xib, xu_n.astype(_BF), (((1,), (1,)), ((), ())),
                            preferred_element_type=_F32)
    s = (c * p).astype(_BF)
    ct = (((0,), (0,)), ((), ()))
    sxi[...] += jax.lax.dot_general(cb, xib, ct, preferred_element_type=_F32)
    svi[...] += jax.lax.dot_general(cb, vib, ct, preferred_element_type=_F32)
    accA[...] += jax.lax.dot_general(s, vib, ct, preferred_element_type=_F32)
    g1[...] += jax.lax.dot_general(vib, xib, ct, preferred_element_type=_F32)
    ones8 = jnp.ones((_BJ, 8), _BF)
    dui8[...] += jax.lax.dot_general(cb, ones8, ct, preferred_element_type=_F32)
    csum[0:1, :] += jnp.sum(xi_t, axis=0, keepdims=True)
    csum[1:2, :] += jnp.sum(vi_t, axis=0, keepdims=True)

    @pl.when(j == _JT - 1)
    def _epilogue():
        xu_nv = xun[...]
        vu = xu_ref[...]
        dui_raw = dui8[...][:, 0:1]
        dui = jnp.maximum(dui_raw, 1.0)
        duj = jnp.maximum(jnp.asarray(_M, _F32) - dui_raw, 1.0)
        sxiv, sviv, av = sxi[...], svi[...], accA[...]
        sxj = csum[0:1, :] - sxiv
        svj = csum[1:2, :] - sviv
        b_pos = jnp.sum(xu_nv * sxiv, axis=1, keepdims=True) / dui
        b_neg = jnp.sum(xu_nv * sxj, axis=1, keepdims=True) / duj
        du = b_pos - b_neg + _ALPHA
        zu1 = av / dui - sviv * (b_neg - _ALPHA) / dui
        xg1 = jax.lax.dot_general(xu_nv.astype(_BF), g1[...].astype(_BF),
                                  (((1,), (0,)), ((), ())),
                                  preferred_element_type=_F32)
        zu2 = (xg1 - av) / duj - svj * (b_pos + _ALPHA) / duj
        dcl = jnp.maximum(du, _CLAMP)
        outu_ref[...] = (zu1 + zu2) / (dcl + dcl)

        w1 = xu_nv / dui
        w2 = xu_nv / duj
        w3 = vu * (b_neg - _ALPHA) / dui
        w4 = vu * (b_pos + _ALPHA) / duj
        w5 = vu / dui
        w6 = vu / duj
        s1 = (-b_neg + _ALPHA) / dui
        s2 = (b_pos + _ALPHA) / duj
        wc_ref[:, 0:128] = w1
        wc_ref[:, 128:256] = w2
        wc_ref[:, 256:384] = w3
        wc_ref[:, 384:512] = w4
        scal = jnp.concatenate(
            [s1, s2, jnp.zeros((_N, 126), _F32)], axis=1)
        wc_ref[:, 512:640] = scal
        ws_ref[:, 0:128] = w5
        ws_ref[:, 128:256] = w6
        g2_ref[...] = jax.lax.dot_general(w2.astype(_BF), vu.astype(_BF),
                                          (((0,), (0,)), ((), ())),
                                          preferred_element_type=_F32)
        aux_ref[...] = jnp.concatenate(
            [jnp.sum(w2, axis=0, keepdims=True),
             jnp.sum(w4, axis=0, keepdims=True),
             jnp.full((1, 128), jnp.sum(s2)),
             jnp.zeros((5, 128), _F32)], axis=0)


def _phase_b_body(c_ref, xit_ref, xu_ref, wc_ref, ws_ref, g2_ref, aux_ref,
                  outi_ref, xun):
    j = pl.program_id(0)

    @pl.when(j == 0)
    def _init():
        xun[...] = _normalize(xu_ref[...])

    c = c_ref[...]
    vi_t = xit_ref[...]
    xi_t = _normalize(vi_t)
    xu_n = xun[...]

    cb = c.astype(_BF)
    p = jax.lax.dot_general(xi_t.astype(_BF), xu_n.astype(_BF),
                            (((1,), (1,)), ((), ())),
                            preferred_element_type=_F32)
    s = (c * p).astype(_BF)
    nrm = (((1,), (0,)), ((), ()))
    segs = jax.lax.dot_general(cb, wc_ref[...].astype(_BF), nrm,
                               preferred_element_type=_F32)
    t56 = jax.lax.dot_general(s, ws_ref[...].astype(_BF), nrm,
                              preferred_element_type=_F32)
    segw1 = segs[:, 0:128]
    segw2 = segs[:, 128:256]
    segw3 = segs[:, 256:384]
    segw4 = segs[:, 384:512]
    segs1 = segs[:, 512:513]
    segs2 = segs[:, 513:514]
    t5 = t56[:, 0:128]
    t6 = t56[:, 128:256]
    aux = aux_ref[...]
    colsum_w2 = aux[0:1, :]
    colsum_w4 = aux[1:2, :]
    rest_b_tot = aux[2:3, 0:1]

    di1 = jnp.sum(xi_t * segw1, axis=1, keepdims=True) + segs1
    rest_x = colsum_w2 - segw2
    di2 = -jnp.sum(xi_t * rest_x, axis=1, keepdims=True) + (rest_b_tot - segs2)
    zi1 = t5 - segw3
    xg2 = jax.lax.dot_general(xi_t.astype(_BF), g2_ref[...].astype(_BF),
                              (((1,), (0,)), ((), ())),
                              preferred_element_type=_F32)
    zi2 = xg2 - t6 - (colsum_w4 - segw4)
    d1 = jnp.maximum(di1, _CLAMP)
    d2 = jnp.maximum(di2, _CLAMP)
    outi_ref[...] = (zi1 + zi2) / (d1 + d2)


_E = 160000
_EPT = _E // 16
_NSTRIPE = 16
_SI = 256
_SE = _SI * _N
_ROWS = (_EPT + 127) // 128
_ZCH = 16000
_RING = 8


def _sc_densify_body(u_hbm, i_hbm, zeros_hbm, ones_hbm, c_hbm,
                     u_v, i_v, stc_v, off_v, idx2d, zero_buf, ones_v, bounce,
                     stripe, sem):
    c = lax.axis_index("c")
    s = lax.axis_index("s")
    pltpu.sync_copy(u_hbm.at[pl.ds(s * _EPT, _EPT)], u_v)
    pltpu.sync_copy(i_hbm.at[pl.ds(s * _EPT, _EPT)], i_v)
    pltpu.sync_copy(zeros_hbm, zero_buf)
    pltpu.sync_copy(ones_hbm, ones_v)

    lanes = jax.lax.iota(jnp.int32, 16)
    half0 = c * _NSTRIPE

    def pre_body(k, _):
        i16 = i_v[pl.ds(k * 16, 16)]
        u16 = u_v[pl.ds(k * 16, 16)]
        stc_v[pl.ds(k * 16, 16)] = (i16 >> 8) - half0
        off_v[pl.ds(k * 16, 16)] = (i16 & 255) * _N + u16
        return 0

    lax.fori_loop(0, _EPT // 16, pre_body, 0)
    dumpv = _SE + lanes

    def stripe_body(st, _):
        base = (c * 4096 + st * _SI) * _N

        def fill_body(g, _):
            stc16 = stc_v[pl.ds(g * 16, 16)]
            off16 = off_v[pl.ds(g * 16, 16)]
            idx2d[g >> 3, pl.ds((g & 7) * 16, 16)] = jnp.where(
                stc16 == st, off16, dumpv)
            return 0

        lax.fori_loop(0, _EPT // 16, fill_body, 0)
        for j in range(1, 8):
            idx2d[_ROWS - 1, pl.ds(j * 16, 16)] = dumpv
        for q in range(2):
            pltpu.sync_copy(zero_buf,
                            stripe.at[pl.ds(s * 2 * _ZCH + q * _ZCH, _ZCH)])
        plsc.subcore_barrier()

        descs = []
        for k in range(_ROWS):
            descs.append(pltpu.async_copy(
                ones_v, stripe.at[idx2d.at[k]], sem, add=True))
            if k >= _RING:
                descs[k - _RING].wait()
        for k in range(_ROWS - _RING, _ROWS):
            descs[k].wait()

        plsc.subcore_barrier()
        short = (st == _NSTRIPE - 1) & (c == 1)

        @pl.when(jnp.logical_not(short))
        def _cout():
            for q in range(2):
                o = s * 2 * _ZCH + q * _ZCH
                pltpu.sync_copy(stripe.at[pl.ds(o, _ZCH)], bounce)
                pltpu.sync_copy(bounce, c_hbm.at[pl.ds(base + o, _ZCH)])

        @pl.when(short)
        def _cout_short():
            pltpu.sync_copy(stripe.at[pl.ds(s * 8000, 8000)],
                            bounce.at[pl.ds(0, 8000)])
            pltpu.sync_copy(bounce.at[pl.ds(0, 8000)],
                            c_hbm.at[pl.ds(base + s * 8000, 8000)])

        plsc.subcore_barrier()
        return 0

    lax.fori_loop(0, _NSTRIPE, stripe_body, 0)


def _densify_counts(u, i):
    builder = pl.kernel(
        _sc_densify_body,
        out_type=jax.ShapeDtypeStruct((_M * _N,), _F32),
        mesh=plsc.VectorSubcoreMesh(core_axis_name="c", subcore_axis_name="s"),
        scratch_types=[
            pltpu.VMEM((_EPT,), jnp.int32),
            pltpu.VMEM((_EPT,), jnp.int32),
            pltpu.VMEM((_EPT,), jnp.int32),
            pltpu.VMEM((_EPT,), jnp.int32),
            pltpu.VMEM((_ROWS, 128), jnp.int32),
            pltpu.VMEM((_ZCH,), _F32),
            pltpu.VMEM((128,), _F32),
            pltpu.VMEM((_ZCH,), _F32),
            pltpu.VMEM_SHARED((_SE + 16,), _F32),
            pltpu.SemaphoreType.DMA,
        ],
    )
    cflat = builder(u, i,
                    jnp.zeros((_ZCH,), _F32), jnp.ones((128,), _F32))
    return cflat.reshape(_M, _N)


def kernel(x, u, i):
    xu = x[:_N]
    xitems = x[_N:]
    c = _densify_counts(u, i)

    a_out = pl.pallas_call(
        _phase_a_body,
        grid=(_JT,),
        in_specs=[
            pl.BlockSpec((_BJ, _N), lambda j: (j, 0)),
            pl.BlockSpec((_BJ, _D), lambda j: (j, 0)),
            pl.BlockSpec((_N, _D), lambda j: (0, 0)),
        ],
        out_specs=[
            pl.BlockSpec((_N, _D), lambda j: (0, 0)),
            pl.BlockSpec((_N, 640), lambda j: (0, 0)),
            pl.BlockSpec((_N, 256), lambda j: (0, 0)),
            pl.BlockSpec((_D, _D), lambda j: (0, 0)),
            pl.BlockSpec((8, 128), lambda j: (0, 0)),
        ],
        out_shape=[
            jax.ShapeDtypeStruct((_N, _D), _F32),
            jax.ShapeDtypeStruct((_N, 640), _F32),
            jax.ShapeDtypeStruct((_N, 256), _F32),
            jax.ShapeDtypeStruct((_D, _D), _F32),
            jax.ShapeDtypeStruct((8, 128), _F32),
        ],
        scratch_shapes=[
            pltpu.VMEM((_N, _D), _F32),
            pltpu.VMEM((_N, _D), _F32),
            pltpu.VMEM((_N, _D), _F32),
            pltpu.VMEM((_D, _D), _F32),
            pltpu.VMEM((_N, 8), _F32),
            pltpu.VMEM((8, _D), _F32),
            pltpu.VMEM((_N, _D), _F32),
        ],
    )(c, xitems, xu)
    out_u, wc, ws, g2, aux = a_out

    out_i = pl.pallas_call(
        _phase_b_body,
        grid=(_JT,),
        in_specs=[
            pl.BlockSpec((_BJ, _N), lambda j: (j, 0)),
            pl.BlockSpec((_BJ, _D), lambda j: (j, 0)),
            pl.BlockSpec((_N, _D), lambda j: (0, 0)),
            pl.BlockSpec((_N, 640), lambda j: (0, 0)),
            pl.BlockSpec((_N, 256), lambda j: (0, 0)),
            pl.BlockSpec((_D, _D), lambda j: (0, 0)),
            pl.BlockSpec((8, 128), lambda j: (0, 0)),
        ],
        out_specs=pl.BlockSpec((_BJ, _D), lambda j: (j, 0)),
        out_shape=jax.ShapeDtypeStruct((_M, _D), _F32),
        scratch_shapes=[pltpu.VMEM((_N, _D), _F32)],
    )(c, xitems, xu, wc, ws, g2, aux)

    return jnp.concatenate([out_u, out_i], axis=0)


# speedup vs baseline: 10.6605x; 1.0498x over previous
import functools
import jax
import jax.numpy as jnp
from jax import lax
from jax.experimental import pallas as pl
from jax.experimental.pallas import tpu as pltpu
from jax.experimental.pallas import tpu_sc as plsc

_N = 2000
_M = 8000
_D = 128
_ALPHA = 1.0
_CLAMP = 1e-06
_BJ = 1000
_JT = _M // _BJ

_F32 = jnp.float32
_BF = jnp.bfloat16


def _normalize(v):
    nrm = jnp.sqrt(jnp.sum(v * v, axis=1, keepdims=True))
    return v / jnp.maximum(nrm, 1e-12)


def _phase_a_body(c_ref, xit_ref, xu_ref,
                  outu_ref, wc_ref, ws_ref, g2_ref, aux_ref,
                  sxi, svi, accA, g1, dui8, csum, xun):
    j = pl.program_id(0)

    @pl.when(j == 0)
    def _init():
        sxi[...] = jnp.zeros_like(sxi)
        svi[...] = jnp.zeros_like(svi)
        accA[...] = jnp.zeros_like(accA)
        g1[...] = jnp.zeros_like(g1)
        dui8[...] = jnp.zeros_like(dui8)
        csum[...] = jnp.zeros_like(csum)
        xun[...] = _normalize(xu_ref[...])

    c = c_ref[...]
    vi_t = xit_ref[...]
    xi_t = _normalize(vi_t)
    xu_n = xun[...]

    cb = c.astype(_BF)
    xib = xi_t.astype(_BF)
    vib = vi_t.astype(_BF)
    p = jax.lax.dot_general(xib, xu_n.astype(_BF), (((1,), (1,)), ((), ())),
                            preferred_element_type=_F32)
    s = (c * p).astype(_BF)
    ct = (((0,), (0,)), ((), ()))
    sxi[...] += jax.lax.dot_general(cb, xib, ct, preferred_element_type=_F32)
    svi[...] += jax.lax.dot_general(cb, vib, ct, preferred_element_type=_F32)
    accA[...] += jax.lax.dot_general(s, vib, ct, preferred_element_type=_F32)
    g1[...] += jax.lax.dot_general(vib, xib, ct, preferred_element_type=_F32)
    ones8 = jnp.ones((_BJ, 8), _BF)
    dui8[...] += jax.lax.dot_general(cb, ones8, ct, preferred_element_type=_F32)
    csum[0:1, :] += jnp.sum(xi_t, axis=0, keepdims=True)
    csum[1:2, :] += jnp.sum(vi_t, axis=0, keepdims=True)

    @pl.when(j == _JT - 1)
    def _epilogue():
        xu_nv = xun[...]
        vu = xu_ref[...]
        dui_raw = dui8[...][:, 0:1]
        dui = jnp.maximum(dui_raw, 1.0)
        duj = jnp.maximum(jnp.asarray(_M, _F32) - dui_raw, 1.0)
        sxiv, sviv, av = sxi[...], svi[...], accA[...]
        sxj = csum[0:1, :] - sxiv
        svj = csum[1:2, :] - sviv
        b_pos = jnp.sum(xu_nv * sxiv, axis=1, keepdims=True) / dui
        b_neg = jnp.sum(xu_nv * sxj, axis=1, keepdims=True) / duj
        du = b_pos - b_neg + _ALPHA
        zu1 = av / dui - sviv * (b_neg - _ALPHA) / dui
        xg1 = jax.lax.dot_general(xu_nv.astype(_BF), g1[...].astype(_BF),
                                  (((1,), (0,)), ((), ())),
                                  preferred_element_type=_F32)
        zu2 = (xg1 - av) / duj - svj * (b_pos + _ALPHA) / duj
        dcl = jnp.maximum(du, _CLAMP)
        outu_ref[...] = (zu1 + zu2) / (dcl + dcl)

        w1 = xu_nv / dui
        w2 = xu_nv / duj
        w3 = vu * (b_neg - _ALPHA) / dui
        w4 = vu * (b_pos + _ALPHA) / duj
        w5 = vu / dui
        w6 = vu / duj
        s1 = (-b_neg + _ALPHA) / dui
        s2 = (b_pos + _ALPHA) / duj
        wc_ref[:, 0:128] = w1
        wc_ref[:, 128:256] = w2
        wc_ref[:, 256:384] = w3
        wc_ref[:, 384:512] = w4
        scal = jnp.concatenate(
            [s1, s2, jnp.zeros((_N, 126), _F32)], axis=1)
        wc_ref[:, 512:640] = scal
        ws_ref[:, 0:128] = w5
        ws_ref[:, 128:256] = w6
        g2_ref[...] = jax.lax.dot_general(w2.astype(_BF), vu.astype(_BF),
                                          (((0,), (0,)), ((), ())),
                                          preferred_element_type=_F32)
        aux_ref[...] = jnp.concatenate(
            [jnp.sum(w2, axis=0, keepdims=True),
             jnp.sum(w4, axis=0, keepdims=True),
             jnp.full((1, 128), jnp.sum(s2)),
             jnp.zeros((5, 128), _F32)], axis=0)


def _phase_b_body(c_ref, xit_ref, xu_ref, wc_ref, ws_ref, g2_ref, aux_ref,
                  outi_ref, xun):
    j = pl.program_id(0)

    @pl.when(j == 0)
    def _init():
        xun[...] = _normalize(xu_ref[...])

    c = c_ref[...]
    vi_t = xit_ref[...]
    xi_t = _normalize(vi_t)
    xu_n = xun[...]

    cb = c.astype(_BF)
    p = jax.lax.dot_general(xi_t.astype(_BF), xu_n.astype(_BF),
                            (((1,), (1,)), ((), ())),
                            preferred_element_type=_F32)
    s = (c * p).astype(_BF)
    nrm = (((1,), (0,)), ((), ()))
    segs = jax.lax.dot_general(cb, wc_ref[...].astype(_BF), nrm,
                               preferred_element_type=_F32)
    t56 = jax.lax.dot_general(s, ws_ref[...].astype(_BF), nrm,
                              preferred_element_type=_F32)
    segw1 = segs[:, 0:128]
    segw2 = segs[:, 128:256]
    segw3 = segs[:, 256:384]
    segw4 = segs[:, 384:512]
    segs1 = segs[:, 512:513]
    segs2 = segs[:, 513:514]
    t5 = t56[:, 0:128]
    t6 = t56[:, 128:256]
    aux = aux_ref[...]
    colsum_w2 = aux[0:1, :]
    colsum_w4 = aux[1:2, :]
    rest_b_tot = aux[2:3, 0:1]

    di1 = jnp.sum(xi_t * segw1, axis=1, keepdims=True) + segs1
    rest_x = colsum_w2 - segw2
    di2 = -jnp.sum(xi_t * rest_x, axis=1, keepdims=True) + (rest_b_tot - segs2)
    zi1 = t5 - segw3
    xg2 = jax.lax.dot_general(xi_t.astype(_BF), g2_ref[...].astype(_BF),
                              (((1,), (0,)), ((), ())),
                              preferred_element_type=_F32)
    zi2 = xg2 - t6 - (colsum_w4 - segw4)
    d1 = jnp.maximum(di1, _CLAMP)
    d2 = jnp.maximum(di2, _CLAMP)
    outi_ref[...] = (zi1 + zi2) / (d1 + d2)


_E = 160000
_EPT = _E // 16
_NSTRIPE = 16
_SI = 256
_SE = _SI * _N
_ROWS = (_EPT + 127) // 128
_ZCH = 16000
_RING = 8


def _sc_densify_body(u_hbm, i_hbm, zeros_hbm, ones_hbm, c_hbm,
                     u_v, i_v, stc_v, off_v, idx2d, zero_buf, ones_v, bounce,
                     stripe, sem, sem2):
    c = lax.axis_index("c")
    s = lax.axis_index("s")
    pltpu.sync_copy(u_hbm.at[pl.ds(s * _EPT, _EPT)], u_v)
    pltpu.sync_copy(i_hbm.at[pl.ds(s * _EPT, _EPT)], i_v)
    pltpu.sync_copy(zeros_hbm, zero_buf)
    pltpu.sync_copy(ones_hbm, ones_v)

    lanes = jax.lax.iota(jnp.int32, 16)
    half0 = c * _NSTRIPE
    neg1 = jnp.full((16,), -1, jnp.int32)

    def pre_body(k, _):
        i16 = i_v[pl.ds(k * 16, 16)]
        u16 = u_v[pl.ds(k * 16, 16)]
        stc_v[pl.ds(k * 16, 16)] = (i16 >> 8) - half0
        off_v[pl.ds(k * 16, 16)] = (i16 & 255) * _N + u16
        return 0

    lax.fori_loop(0, _EPT // 16, pre_body, 0)
    for g in range(_EPT // 16, _ROWS * 8):
        stc_v[pl.ds(g * 16, 16)] = neg1

    dumpv = _SE + lanes

    def fill(stw):
        def fill_body(k, _):
            for g in range(8):
                stc16 = stc_v[pl.ds(k * 128 + g * 16, 16)]
                off16 = off_v[pl.ds(k * 128 + g * 16, 16)]
                idx2d[k, pl.ds(g * 16, 16)] = jnp.where(
                    stc16 == stw, off16, dumpv)
            return 0

        lax.fori_loop(0, _ROWS, fill_body, 0)

    def zero_own():
        for q in range(2):
            pltpu.sync_copy(zero_buf,
                            stripe.at[pl.ds(s * 2 * _ZCH + q * _ZCH, _ZCH)])

    fill(0)
    zero_own()

    def stripe_body(st, _):
        base = (c * 4096 + st * _SI) * _N
        plsc.subcore_barrier()

        descs = []
        for k in range(_ROWS):
            descs.append(pltpu.async_copy(
                ones_v, stripe.at[idx2d.at[k]], sem, add=True))
            if k >= _RING:
                descs[k - _RING].wait()
        for k in range(_ROWS - _RING, _ROWS):
            descs[k].wait()

        plsc.subcore_barrier()
        short = (st == _NSTRIPE - 1) & (c == 1)

        @pl.when(jnp.logical_not(short))
        def _cout():
            o = s * 2 * _ZCH
            for q in range(2):
                pltpu.sync_copy(stripe.at[pl.ds(o + q * _ZCH, _ZCH)],
                                bounce.at[pl.ds(q * _ZCH, _ZCH)])
            for q in range(2):
                pltpu.sync_copy(bounce.at[pl.ds(q * _ZCH, _ZCH)],
                                c_hbm.at[pl.ds(base + o + q * _ZCH, _ZCH)])
            fill(st + 1)
            zero_own()

        @pl.when(short)
        def _cout_short():
            pltpu.sync_copy(stripe.at[pl.ds(s * 8000, 8000)],
                            bounce.at[pl.ds(0, 8000)])
            pltpu.sync_copy(bounce.at[pl.ds(0, 8000)],
                            c_hbm.at[pl.ds(base + s * 8000, 8000)])

        return 0

    lax.fori_loop(0, _NSTRIPE, stripe_body, 0)


def _densify_counts(u, i):
    builder = pl.kernel(
        _sc_densify_body,
        out_type=jax.ShapeDtypeStruct((_M * _N,), _F32),
        mesh=plsc.VectorSubcoreMesh(core_axis_name="c", subcore_axis_name="s"),
        scratch_types=[
            pltpu.VMEM((_EPT,), jnp.int32),
            pltpu.VMEM((_EPT,), jnp.int32),
            pltpu.VMEM((_ROWS * 128,), jnp.int32),
            pltpu.VMEM((_ROWS * 128,), jnp.int32),
            pltpu.VMEM((_ROWS, 128), jnp.int32),
            pltpu.VMEM((_ZCH,), _F32),
            pltpu.VMEM((128,), _F32),
            pltpu.VMEM((2 * _ZCH,), _F32),
            pltpu.VMEM_SHARED((_SE + 16,), _F32),
            pltpu.SemaphoreType.DMA,
            pltpu.SemaphoreType.DMA,
        ],
    )
    cflat = builder(u, i,
                    jnp.zeros((_ZCH,), _F32), jnp.ones((128,), _F32))
    return cflat.reshape(_M, _N)


def kernel(x, u, i):
    xu = x[:_N]
    xitems = x[_N:]
    c = _densify_counts(u, i)

    a_out = pl.pallas_call(
        _phase_a_body,
        grid=(_JT,),
        in_specs=[
            pl.BlockSpec((_BJ, _N), lambda j: (j, 0)),
            pl.BlockSpec((_BJ, _D), lambda j: (j, 0)),
            pl.BlockSpec((_N, _D), lambda j: (0, 0)),
        ],
        out_specs=[
            pl.BlockSpec((_N, _D), lambda j: (0, 0)),
            pl.BlockSpec((_N, 640), lambda j: (0, 0)),
            pl.BlockSpec((_N, 256), lambda j: (0, 0)),
            pl.BlockSpec((_D, _D), lambda j: (0, 0)),
            pl.BlockSpec((8, 128), lambda j: (0, 0)),
        ],
        out_shape=[
            jax.ShapeDtypeStruct((_N, _D), _F32),
            jax.ShapeDtypeStruct((_N, 640), _F32),
            jax.ShapeDtypeStruct((_N, 256), _F32),
            jax.ShapeDtypeStruct((_D, _D), _F32),
            jax.ShapeDtypeStruct((8, 128), _F32),
        ],
        scratch_shapes=[
            pltpu.VMEM((_N, _D), _F32),
            pltpu.VMEM((_N, _D), _F32),
            pltpu.VMEM((_N, _D), _F32),
            pltpu.VMEM((_D, _D), _F32),
            pltpu.VMEM((_N, 8), _F32),
            pltpu.VMEM((8, _D), _F32),
            pltpu.VMEM((_N, _D), _F32),
        ],
    )(c, xitems, xu)
    out_u, wc, ws, g2, aux = a_out

    out_i = pl.pallas_call(
        _phase_b_body,
        grid=(_JT,),
        in_specs=[
            pl.BlockSpec((_BJ, _N), lambda j: (j, 0)),
            pl.BlockSpec((_BJ, _D), lambda j: (j, 0)),
            pl.BlockSpec((_N, _D), lambda j: (0, 0)),
            pl.BlockSpec((_N, 640), lambda j: (0, 0)),
            pl.BlockSpec((_N, 256), lambda j: (0, 0)),
            pl.BlockSpec((_D, _D), lambda j: (0, 0)),
            pl.BlockSpec((8, 128), lambda j: (0, 0)),
        ],
        out_specs=pl.BlockSpec((_BJ, _D), lambda j: (j, 0)),
        out_shape=jax.ShapeDtypeStruct((_M, _D), _F32),
        scratch_shapes=[pltpu.VMEM((_N, _D), _F32)],
    )(c, xitems, xu, wc, ws, g2, aux)

    return jnp.concatenate([out_u, out_i], axis=0)


# speedup vs baseline: 10.7007x; 1.0038x over previous
import functools
import jax
import jax.numpy as jnp
from jax import lax
from jax.experimental import pallas as pl
from jax.experimental.pallas import tpu as pltpu
from jax.experimental.pallas import tpu_sc as plsc

_N = 2000
_M = 8000
_D = 128
_ALPHA = 1.0
_CLAMP = 1e-06
_BJ = 1000
_JT = _M // _BJ

_F32 = jnp.float32
_BF = jnp.bfloat16


def _normalize(v):
    nrm = jnp.sqrt(jnp.sum(v * v, axis=1, keepdims=True))
    return v / jnp.maximum(nrm, 1e-12)


def _phase_a_body(c_ref, xit_ref, xu_ref,
                  outu_ref, wc_ref, ws_ref, g2_ref, aux_ref,
                  sxi, svi, accA, g1, dui8, csum, xun):
    j = pl.program_id(0)

    @pl.when(j == 0)
    def _init():
        sxi[...] = jnp.zeros_like(sxi)
        svi[...] = jnp.zeros_like(svi)
        accA[...] = jnp.zeros_like(accA)
        g1[...] = jnp.zeros_like(g1)
        dui8[...] = jnp.zeros_like(dui8)
        csum[...] = jnp.zeros_like(csum)
        xun[...] = _normalize(xu_ref[...])

    c = c_ref[...]
    vi_t = xit_ref[...]
    xi_t = _normalize(vi_t)
    xu_n = xun[...]

    cb = c.astype(_BF)
    xib = xi_t.astype(_BF)
    vib = vi_t.astype(_BF)
    p = jax.lax.dot_general(xib, xu_n.astype(_BF), (((1,), (1,)), ((), ())),
                            preferred_element_type=_F32)
    s = (c * p).astype(_BF)
    ct = (((0,), (0,)), ((), ()))
    sxi[...] += jax.lax.dot_general(cb, xib, ct, preferred_element_type=_F32)
    svi[...] += jax.lax.dot_general(cb, vib, ct, preferred_element_type=_F32)
    accA[...] += jax.lax.dot_general(s, vib, ct, preferred_element_type=_F32)
    g1[...] += jax.lax.dot_general(vib, xib, ct, preferred_element_type=_F32)
    ones8 = jnp.ones((_BJ, 8), _BF)
    dui8[...] += jax.lax.dot_general(cb, ones8, ct, preferred_element_type=_F32)
    csum[0:1, :] += jnp.sum(xi_t, axis=0, keepdims=True)
    csum[1:2, :] += jnp.sum(vi_t, axis=0, keepdims=True)

    @pl.when(j == _JT - 1)
    def _epilogue():
        xu_nv = xun[...]
        vu = xu_ref[...]
        dui_raw = dui8[...][:, 0:1]
        dui = jnp.maximum(dui_raw, 1.0)
        duj = jnp.maximum(jnp.asarray(_M, _F32) - dui_raw, 1.0)
        sxiv, sviv, av = sxi[...], svi[...], accA[...]
        sxj = csum[0:1, :] - sxiv
        svj = csum[1:2, :] - sviv
        b_pos = jnp.sum(xu_nv * sxiv, axis=1, keepdims=True) / dui
        b_neg = jnp.sum(xu_nv * sxj, axis=1, keepdims=True) / duj
        du = b_pos - b_neg + _ALPHA
        zu1 = av / dui - sviv * (b_neg - _ALPHA) / dui
        xg1 = jax.lax.dot_general(xu_nv.astype(_BF), g1[...].astype(_BF),
                                  (((1,), (0,)), ((), ())),
                                  preferred_element_type=_F32)
        zu2 = (xg1 - av) / duj - svj * (b_pos + _ALPHA) / duj
        dcl = jnp.maximum(du, _CLAMP)
        outu_ref[...] = (zu1 + zu2) / (dcl + dcl)

        w1 = xu_nv / dui
        w2 = xu_nv / duj
        w3 = vu * (b_neg - _ALPHA) / dui
        w4 = vu * (b_pos + _ALPHA) / duj
        w5 = vu / dui
        w6 = vu / duj
        s1 = (-b_neg + _ALPHA) / dui
        s2 = (b_pos + _ALPHA) / duj
        wc_ref[:, 0:128] = w1
        wc_ref[:, 128:256] = w2
        wc_ref[:, 256:384] = w3
        wc_ref[:, 384:512] = w4
        scal = jnp.concatenate(
            [s1, s2, jnp.zeros((_N, 126), _F32)], axis=1)
        wc_ref[:, 512:640] = scal
        ws_ref[:, 0:128] = w5
        ws_ref[:, 128:256] = w6
        g2_ref[...] = jax.lax.dot_general(w2.astype(_BF), vu.astype(_BF),
                                          (((0,), (0,)), ((), ())),
                                          preferred_element_type=_F32)
        aux_ref[...] = jnp.concatenate(
            [jnp.sum(w2, axis=0, keepdims=True),
             jnp.sum(w4, axis=0, keepdims=True),
             jnp.full((1, 128), jnp.sum(s2)),
             jnp.zeros((5, 128), _F32)], axis=0)


def _phase_b_body(c_ref, xit_ref, xu_ref, wc_ref, ws_ref, g2_ref, aux_ref,
                  outi_ref, xun):
    j = pl.program_id(0)

    @pl.when(j == 0)
    def _init():
        xun[...] = _normalize(xu_ref[...])

    c = c_ref[...]
    vi_t = xit_ref[...]
    xi_t = _normalize(vi_t)
    xu_n = xun[...]

    cb = c.astype(_BF)
    p = jax.lax.dot_general(xi_t.astype(_BF), xu_n.astype(_BF),
                            (((1,), (1,)), ((), ())),
                            preferred_element_type=_F32)
    s = (c * p).astype(_BF)
    nrm = (((1,), (0,)), ((), ()))
    segs = jax.lax.dot_general(cb, wc_ref[...].astype(_BF), nrm,
                               preferred_element_type=_F32)
    t56 = jax.lax.dot_general(s, ws_ref[...].astype(_BF), nrm,
                              preferred_element_type=_F32)
    segw1 = segs[:, 0:128]
    segw2 = segs[:, 128:256]
    segw3 = segs[:, 256:384]
    segw4 = segs[:, 384:512]
    segs1 = segs[:, 512:513]
    segs2 = segs[:, 513:514]
    t5 = t56[:, 0:128]
    t6 = t56[:, 128:256]
    aux = aux_ref[...]
    colsum_w2 = aux[0:1, :]
    colsum_w4 = aux[1:2, :]
    rest_b_tot = aux[2:3, 0:1]

    di1 = jnp.sum(xi_t * segw1, axis=1, keepdims=True) + segs1
    rest_x = colsum_w2 - segw2
    di2 = -jnp.sum(xi_t * rest_x, axis=1, keepdims=True) + (rest_b_tot - segs2)
    zi1 = t5 - segw3
    xg2 = jax.lax.dot_general(xi_t.astype(_BF), g2_ref[...].astype(_BF),
                              (((1,), (0,)), ((), ())),
                              preferred_element_type=_F32)
    zi2 = xg2 - t6 - (colsum_w4 - segw4)
    d1 = jnp.maximum(di1, _CLAMP)
    d2 = jnp.maximum(di2, _CLAMP)
    outi_ref[...] = (zi1 + zi2) / (d1 + d2)


_E = 160000
_EPT = _E // 16
_NSTRIPE = 16
_SI = 256
_SE = _SI * _N
_ROWS = (_EPT + 127) // 128
_ZCH = 16000
_RING = 8


def _sc_densify_body(u_hbm, i_hbm, zeros_hbm, ones_hbm, c_hbm,
                     u_v, i_v, stc_v, off_v, idx2d, zero_buf, ones_v, bounce,
                     stripe, sem, sem2):
    c = lax.axis_index("c")
    s = lax.axis_index("s")
    pltpu.sync_copy(u_hbm.at[pl.ds(s * _EPT, _EPT)], u_v)
    pltpu.sync_copy(i_hbm.at[pl.ds(s * _EPT, _EPT)], i_v)
    pltpu.sync_copy(zeros_hbm, zero_buf)
    pltpu.sync_copy(ones_hbm, ones_v)

    lanes = jax.lax.iota(jnp.int32, 16)
    half0 = c * _NSTRIPE
    neg1 = jnp.full((16,), -1, jnp.int32)

    def pre_body(k, _):
        i16 = i_v[pl.ds(k * 16, 16)]
        u16 = u_v[pl.ds(k * 16, 16)]
        stc_v[pl.ds(k * 16, 16)] = (i16 >> 8) - half0
        off_v[pl.ds(k * 16, 16)] = (i16 & 255) * _N + u16
        return 0

    lax.fori_loop(0, _EPT // 16, pre_body, 0)
    for g in range(_EPT // 16, _ROWS * 8):
        stc_v[pl.ds(g * 16, 16)] = neg1

    dumpv = _SE + lanes

    def fill(stw):
        def fill_body(k, _):
            for g in range(8):
                stc16 = stc_v[pl.ds(k * 128 + g * 16, 16)]
                off16 = off_v[pl.ds(k * 128 + g * 16, 16)]
                idx2d[k, pl.ds(g * 16, 16)] = jnp.where(
                    stc16 == stw, off16, dumpv)
            return 0

        lax.fori_loop(0, _ROWS, fill_body, 0)

    def zero_own():
        for q in range(2):
            pltpu.sync_copy(zero_buf,
                            stripe.at[pl.ds(s * 2 * _ZCH + q * _ZCH, _ZCH)])

    fill(0)
    zero_own()

    def stripe_body(st, _):
        base = (c * 4096 + st * _SI) * _N
        plsc.subcore_barrier()

        descs = []
        for k in range(_ROWS):
            descs.append(pltpu.async_copy(
                ones_v, stripe.at[idx2d.at[k]], sem, add=True))
            if k >= _RING:
                descs[k - _RING].wait()
        for k in range(_ROWS - _RING, _ROWS):
            descs[k].wait()

        plsc.subcore_barrier()
        short = (st == _NSTRIPE - 1) & (c == 1)

        @pl.when(jnp.logical_not(short))
        def _cout():
            o = s * 2 * _ZCH
            pltpu.sync_copy(stripe.at[pl.ds(o, 2 * _ZCH)], bounce)
            pltpu.sync_copy(bounce, c_hbm.at[pl.ds(base + o, 2 * _ZCH)])
            fill(st + 1)
            zero_own()

        @pl.when(short)
        def _cout_short():
            pltpu.sync_copy(stripe.at[pl.ds(s * 8000, 8000)],
                            bounce.at[pl.ds(0, 8000)])
            pltpu.sync_copy(bounce.at[pl.ds(0, 8000)],
                            c_hbm.at[pl.ds(base + s * 8000, 8000)])

        return 0

    lax.fori_loop(0, _NSTRIPE, stripe_body, 0)


def _densify_counts(u, i):
    builder = pl.kernel(
        _sc_densify_body,
        out_type=jax.ShapeDtypeStruct((_M * _N,), _F32),
        mesh=plsc.VectorSubcoreMesh(core_axis_name="c", subcore_axis_name="s"),
        scratch_types=[
            pltpu.VMEM((_EPT,), jnp.int32),
            pltpu.VMEM((_EPT,), jnp.int32),
            pltpu.VMEM((_ROWS * 128,), jnp.int32),
            pltpu.VMEM((_ROWS * 128,), jnp.int32),
            pltpu.VMEM((_ROWS, 128), jnp.int32),
            pltpu.VMEM((_ZCH,), _F32),
            pltpu.VMEM((128,), _F32),
            pltpu.VMEM((2 * _ZCH,), _F32),
            pltpu.VMEM_SHARED((_SE + 16,), _F32),
            pltpu.SemaphoreType.DMA,
            pltpu.SemaphoreType.DMA,
        ],
    )
    cflat = builder(u, i,
                    jnp.zeros((_ZCH,), _F32), jnp.ones((128,), _F32))
    return cflat.reshape(_M, _N)


def kernel(x, u, i):
    xu = x[:_N]
    xitems = x[_N:]
    c = _densify_counts(u, i)

    a_out = pl.pallas_call(
        _phase_a_body,
        grid=(_JT,),
        in_specs=[
            pl.BlockSpec((_BJ, _N), lambda j: (j, 0)),
            pl.BlockSpec((_BJ, _D), lambda j: (j, 0)),
            pl.BlockSpec((_N, _D), lambda j: (0, 0)),
        ],
        out_specs=[
            pl.BlockSpec((_N, _D), lambda j: (0, 0)),
            pl.BlockSpec((_N, 640), lambda j: (0, 0)),
            pl.BlockSpec((_N, 256), lambda j: (0, 0)),
            pl.BlockSpec((_D, _D), lambda j: (0, 0)),
            pl.BlockSpec((8, 128), lambda j: (0, 0)),
        ],
        out_shape=[
            jax.ShapeDtypeStruct((_N, _D), _F32),
            jax.ShapeDtypeStruct((_N, 640), _F32),
            jax.ShapeDtypeStruct((_N, 256), _F32),
            jax.ShapeDtypeStruct((_D, _D), _F32),
            jax.ShapeDtypeStruct((8, 128), _F32),
        ],
        scratch_shapes=[
            pltpu.VMEM((_N, _D), _F32),
            pltpu.VMEM((_N, _D), _F32),
            pltpu.VMEM((_N, _D), _F32),
            pltpu.VMEM((_D, _D), _F32),
            pltpu.VMEM((_N, 8), _F32),
            pltpu.VMEM((8, _D), _F32),
            pltpu.VMEM((_N, _D), _F32),
        ],
    )(c, xitems, xu)
    out_u, wc, ws, g2, aux = a_out

    out_i = pl.pallas_call(
        _phase_b_body,
        grid=(_JT,),
        in_specs=[
            pl.BlockSpec((_BJ, _N), lambda j: (j, 0)),
            pl.BlockSpec((_BJ, _D), lambda j: (j, 0)),
            pl.BlockSpec((_N, _D), lambda j: (0, 0)),
            pl.BlockSpec((_N, 640), lambda j: (0, 0)),
            pl.BlockSpec((_N, 256), lambda j: (0, 0)),
            pl.BlockSpec((_D, _D), lambda j: (0, 0)),
            pl.BlockSpec((8, 128), lambda j: (0, 0)),
        ],
        out_specs=pl.BlockSpec((_BJ, _D), lambda j: (j, 0)),
        out_shape=jax.ShapeDtypeStruct((_M, _D), _F32),
        scratch_shapes=[pltpu.VMEM((_N, _D), _F32)],
    )(c, xitems, xu, wc, ws, g2, aux)

    return jnp.concatenate([out_u, out_i], axis=0)
